# baseline (speedup 1.0000x reference)
import jax
import jax.numpy as jnp
from jax import lax
from jax.experimental import pallas as pl
from jax.experimental.pallas import tpu as pltpu
from jax.experimental.pallas import tpu_sc as plsc

NC = 2
NS = 16
L = 16
NW = NC * NS
BATCH = 16384
D = 32
BPW = BATCH // NW
CHUNK = 128
NCHUNK = BPW // CHUNK
GROUPS = BPW // L


def _sc_body(uids_hbm, aids_hbm, utab_hbm, atab_hbm, out_hbm,
             uidx_v, aidx_v, urows_v, arows_v, out_v, sem):
    wid = lax.axis_index("s") * NC + lax.axis_index("c")
    base = wid * BPW

    pltpu.sync_copy(uids_hbm.at[pl.ds(wid * NCHUNK, NCHUNK)], uidx_v)
    pltpu.sync_copy(aids_hbm.at[pl.ds(wid * NCHUNK, NCHUNK)], aidx_v)

    copies = []
    for j in range(NCHUNK):
        copies.append(pltpu.make_async_copy(
            utab_hbm.at[uidx_v.at[j]], urows_v.at[pl.ds(j * CHUNK, CHUNK)], sem))
        copies.append(pltpu.make_async_copy(
            atab_hbm.at[aidx_v.at[j]], arows_v.at[pl.ds(j * CHUNK, CHUNK)], sem))
    for c in copies:
        c.start()
    for c in copies:
        c.wait()

    iota = lax.iota(jnp.int32, L)
    zeros = jnp.zeros((L,), jnp.int32)
    ones = jnp.ones((L,), jnp.int32)

    @pl.loop(0, GROUPS)
    def _group(g):
        rid = g * L + iota
        acc = jnp.zeros((L,), jnp.float32)
        for d in range(D):
            cid = jnp.full((L,), d, jnp.int32)
            u = plsc.load_gather(urows_v, [rid, cid])
            a = plsc.load_gather(arows_v, [rid, cid])
            acc = acc + u * a
        p = 1.0 / (1.0 + jnp.exp(-acc))
        plsc.store_scatter(out_v, [rid, zeros], 1.0 - p)
        plsc.store_scatter(out_v, [rid, ones], p)

    pltpu.sync_copy(out_v, out_hbm.at[pl.ds(base, BPW)])


def kernel(user_ids, ad_ids, user_table, ad_table):
    uids = user_ids.astype(jnp.int32).reshape(NW * NCHUNK, CHUNK)
    aids = ad_ids.astype(jnp.int32).reshape(NW * NCHUNK, CHUNK)
    mesh = plsc.VectorSubcoreMesh(core_axis_name="c", subcore_axis_name="s",
                                  num_cores=NC, num_subcores=NS)
    f = pl.kernel(
        _sc_body,
        out_type=jax.ShapeDtypeStruct((BATCH, 2), jnp.float32),
        mesh=mesh,
        compiler_params=pltpu.CompilerParams(
            needs_layout_passes=False, use_tc_tiling_on_sc=False),
        scratch_types=[
            pltpu.VMEM((NCHUNK, CHUNK), jnp.int32),
            pltpu.VMEM((NCHUNK, CHUNK), jnp.int32),
            pltpu.VMEM((BPW, D), jnp.float32),
            pltpu.VMEM((BPW, D), jnp.float32),
            pltpu.VMEM((BPW, 2), jnp.float32),
            pltpu.SemaphoreType.DMA,
        ],
    )
    return f(uids, aids, user_table, ad_table)
